# baseline (speedup 1.0000x reference)
import functools

import jax
import jax.numpy as jnp
from jax import lax
from jax.experimental import pallas as pl
from jax.experimental.pallas import tpu as pltpu
from jax.experimental.pallas import tpu_sc as plsc

DIM = 768
NC, NS = 2, 16
NW = NC * NS
L = 16
QG = 2


@functools.partial(jax.jit, static_argnums=(2,))
def _sc_gather(W, idx, n_total):
    n_per_w = n_total // NW
    n_grps = n_per_w // L
    mesh = plsc.VectorSubcoreMesh(core_axis_name="c", subcore_axis_name="s")

    @functools.partial(
        pl.kernel,
        mesh=mesh,
        out_type=jax.ShapeDtypeStruct((n_total, DIM), jnp.float32),
        scratch_types=[
            pltpu.VMEM((n_per_w,), jnp.int32),
            pltpu.SemaphoreType.DMA,
        ],
    )
    def k(W_hbm, idx_hbm, out_hbm, idx_v, sem):
        wid = lax.axis_index("s") * NC + lax.axis_index("c")
        base = wid * n_per_w
        pltpu.sync_copy(idx_hbm.at[pl.ds(base, n_per_w)], idx_v)

        def issue_group(g):
            vec = idx_v[pl.ds(g * L, L)]
            for l in range(L):
                pltpu.async_copy(
                    W_hbm.at[pl.ds(vec[l], 1)],
                    out_hbm.at[pl.ds(base + g * L + l, 1)], sem)

        def drain_group():
            for _ in range(L):
                pltpu.make_async_copy(
                    W_hbm.at[pl.ds(0, 1)], out_hbm.at[pl.ds(base, 1)],
                    sem).wait()

        @pl.loop(0, QG)
        def _prime(g):
            issue_group(g)

        @pl.loop(QG, n_grps)
        def _steady(g):
            drain_group()
            issue_group(g)

        @pl.loop(0, QG)
        def _drain(g):
            drain_group()

    return k(W, idx)


def kernel(x, W):
    B, S = x.shape
    n_total = B * S
    out = _sc_gather(W, x.reshape(n_total), n_total)
    return out.reshape(B, S, DIM)


# speedup vs baseline: 42.0202x; 42.0202x over previous
import functools

import jax
import jax.numpy as jnp
from jax import lax
from jax.experimental import pallas as pl
from jax.experimental.pallas import tpu as pltpu
from jax.experimental.pallas import tpu_sc as plsc

DIM = 768
NC, NS = 2, 16
NW = NC * NS
K = 40
NBUF = 2


@functools.partial(jax.jit, static_argnums=(2,))
def _sc_gather(W, idx, n_total):
    n_per_w = n_total // NW
    n_chunks = n_per_w // K
    mesh = plsc.VectorSubcoreMesh(core_axis_name="c", subcore_axis_name="s")

    @functools.partial(
        pl.kernel,
        mesh=mesh,
        out_type=jax.ShapeDtypeStruct((n_total, DIM), jnp.float32),
        scratch_types=[
            pltpu.VMEM((n_per_w,), jnp.int32),
            pltpu.VMEM((NBUF, K, DIM), jnp.float32),
            pltpu.VMEM_SHARED((NS, K, DIM), jnp.float32),
            [pltpu.SemaphoreType.DMA] * NBUF,
            [pltpu.SemaphoreType.DMA] * NBUF,
            pltpu.SemaphoreType.DMA,
        ],
    )
    def k(W_hbm, idx_hbm, out_hbm, idx_v, rows_v, spm, gsems, ssems, xsem):
        sid = lax.axis_index("s")
        wid = sid * NC + lax.axis_index("c")
        base = wid * n_per_w
        pltpu.sync_copy(idx_hbm.at[pl.ds(base, n_per_w)], idx_v)

        def gather(c, b):
            pltpu.async_copy(
                W_hbm.at[idx_v.at[pl.ds(c * K, K)]], rows_v.at[b], gsems[b])

        def wait_gather(c, b):
            pltpu.make_async_copy(
                W_hbm.at[idx_v.at[pl.ds(c * K, K)]], rows_v.at[b],
                gsems[b]).wait()

        def store_direct(c, b):
            pltpu.async_copy(
                rows_v.at[b], out_hbm.at[pl.ds(base + c * K, K)], ssems[b])

        def wait_store_direct(c, b):
            pltpu.make_async_copy(
                rows_v.at[b], out_hbm.at[pl.ds(base + c * K, K)],
                ssems[b]).wait()

        def store_spmem(c, b):
            pltpu.async_copy(rows_v.at[b], spm.at[sid], xsem)
            pltpu.make_async_copy(rows_v.at[b], spm.at[sid], xsem).wait()
            pltpu.async_copy(
                spm.at[sid], out_hbm.at[pl.ds(base + c * K, K)], ssems[b])

        def wait_store_spmem(c, b):
            pltpu.make_async_copy(
                spm.at[sid], out_hbm.at[pl.ds(base + c * K, K)],
                ssems[b]).wait()

        for b in range(NBUF):
            gather(b, b)

        @pl.loop(0, n_chunks, step=NBUF)
        def _grp(j):
            wait_gather(j, 0)
            store_direct(j, 0)
            wait_gather(j + 1, 1)
            store_spmem(j + 1, 1)

            @pl.when(j + NBUF < n_chunks)
            def _():
                wait_store_direct(j, 0)
                gather(j + NBUF, 0)
                wait_store_spmem(j + 1, 1)
                gather(j + NBUF + 1, 1)

        wait_store_direct(n_chunks - 2, 0)
        wait_store_spmem(n_chunks - 1, 1)

    return k(W, idx)


def kernel(x, W):
    B, S = x.shape
    n_total = B * S
    out = _sc_gather(W, x.reshape(n_total), n_total)
    return out.reshape(B, S, DIM)


# speedup vs baseline: 42.1173x; 1.0023x over previous
import functools

import jax
import jax.numpy as jnp
from jax import lax
from jax.experimental import pallas as pl
from jax.experimental.pallas import tpu as pltpu
from jax.experimental.pallas import tpu_sc as plsc

DIM = 768
NC, NS = 2, 16
NW = NC * NS
K = 40
NBUF = 2


@functools.partial(jax.jit, static_argnums=(2,))
def _sc_gather(W, idx, n_total):
    n_per_w = n_total // NW
    n_chunks = n_per_w // K
    mesh = plsc.VectorSubcoreMesh(core_axis_name="c", subcore_axis_name="s")

    @functools.partial(
        pl.kernel,
        mesh=mesh,
        out_type=jax.ShapeDtypeStruct((n_total, DIM), jnp.float32),
        scratch_types=[
            pltpu.VMEM((n_per_w,), jnp.int32),
            pltpu.VMEM((NBUF, K, DIM), jnp.float32),
            pltpu.VMEM_SHARED((NS, K, DIM), jnp.float32),
            [pltpu.SemaphoreType.DMA] * NBUF,
            [pltpu.SemaphoreType.DMA] * NBUF,
            pltpu.SemaphoreType.DMA,
        ],
    )
    def k(W_hbm, idx_hbm, out_hbm, idx_v, rows_v, spm, gsems, ssems, xsem):
        sid = lax.axis_index("s")
        wid = sid * NC + lax.axis_index("c")
        base = wid * n_per_w
        pltpu.sync_copy(idx_hbm.at[pl.ds(base, n_per_w)], idx_v)

        def gather(c, b):
            pltpu.async_copy(
                W_hbm.at[idx_v.at[pl.ds(c * K, K)]], rows_v.at[b], gsems[b])

        def wait_gather(c, b):
            pltpu.make_async_copy(
                W_hbm.at[idx_v.at[pl.ds(c * K, K)]], rows_v.at[b],
                gsems[b]).wait()

        def store_direct(c, b):
            pltpu.async_copy(
                rows_v.at[b], out_hbm.at[pl.ds(base + c * K, K)], ssems[b])

        def wait_store_direct(c, b):
            pltpu.make_async_copy(
                rows_v.at[b], out_hbm.at[pl.ds(base + c * K, K)],
                ssems[b]).wait()

        def store_spmem(c, b):
            pltpu.async_copy(rows_v.at[b], spm.at[sid], xsem)
            pltpu.make_async_copy(rows_v.at[b], spm.at[sid], xsem).wait()
            pltpu.async_copy(
                spm.at[sid], out_hbm.at[pl.ds(base + c * K, K)], ssems[b])

        def wait_store_spmem(c, b):
            pltpu.make_async_copy(
                spm.at[sid], out_hbm.at[pl.ds(base + c * K, K)],
                ssems[b]).wait()

        for b in range(NBUF):
            gather(b, b)

        @pl.loop(0, n_chunks, step=NBUF)
        def _grp(j):
            wait_gather(j, 0)
            store_direct(j, 0)
            wait_gather(j + 1, 1)

            @pl.when(j > 0)
            def _():
                wait_store_spmem(j - 1, 1)

            store_spmem(j + 1, 1)

            @pl.when(j + NBUF < n_chunks)
            def _():
                wait_store_direct(j, 0)
                gather(j + NBUF, 0)
                gather(j + NBUF + 1, 1)

        wait_store_direct(n_chunks - 2, 0)
        wait_store_spmem(n_chunks - 1, 1)

    return k(W, idx)


def kernel(x, W):
    B, S = x.shape
    n_total = B * S
    out = _sc_gather(W, x.reshape(n_total), n_total)
    return out.reshape(B, S, DIM)
